# baseline (speedup 1.0000x reference)
import jax
import jax.numpy as jnp
from jax import lax
from jax.experimental import pallas as pl
from jax.experimental.pallas import tpu as pltpu
from jax.experimental.pallas import tpu_sc as plsc

N_NODES = 10000
N_PAD = 10240
D_FEAT = 128
N_LAYERS = 3

NC = 2
NS = 16
LANES = 16
DH = D_FEAT // NC
QF = DH // LANES

EDGE_BLK = 128
CHUNK_BLKS = 8
NB = 160
NCH = NB // CHUNK_BLKS
E_PAD = NS * NB * EDGE_BLK
ROWS_PER_TILE = N_PAD // NS
OUT_CHUNK = 128


def _sc_body(h0_hbm, e2_hbm, wl_hbm, out_hbm,
             h_a, h_b, e2_c, wl_c, rows, rows2, hbuf, gsem, ssem):
    c = lax.axis_index("c")
    s = lax.axis_index("s")
    r0 = s * ROWS_PER_TILE

    pltpu.sync_copy(h0_hbm.at[c, pl.ds(r0, ROWS_PER_TILE)],
                    h_a.at[pl.ds(r0, ROWS_PER_TILE)])

    bufs = [h_a, h_b]
    for k in range(N_LAYERS):
        gsrc = bufs[k % 2]
        acc = bufs[(k + 1) % 2]
        pltpu.sync_copy(h0_hbm.at[c, pl.ds(r0, ROWS_PER_TILE)],
                        acc.at[pl.ds(r0, ROWS_PER_TILE)])
        plsc.subcore_barrier()

        @pl.loop(0, NCH)
        def _chunk(ch):
            pltpu.sync_copy(e2_hbm.at[s, ch], e2_c)
            pltpu.sync_copy(wl_hbm.at[k, s, ch], wl_c)

            rbufs = [rows, rows2]
            pltpu.async_copy(gsrc.at[e2_c.at[0, 0]], rbufs[0], gsem)
            for b in range(CHUNK_BLKS):
                rb = rbufs[b % 2]
                ro = rbufs[1 - b % 2]
                pltpu.make_async_copy(
                    gsrc.at[e2_c.at[0, b]], rb, gsem).wait()

                if b + 1 < CHUNK_BLKS:
                    if b >= 1:
                        pltpu.make_async_copy(
                            ro, acc.at[e2_c.at[1, b - 1]], ssem).wait()
                    pltpu.async_copy(gsrc.at[e2_c.at[0, b + 1]], ro, gsem)

                @pl.loop(0, EDGE_BLK, step=LANES)
                def _edge_group(g):
                    sv16 = (wl_c[0, b, pl.ds(g, LANES)] *
                            wl_c[1, b, pl.ds(g, LANES)])
                    for j in range(LANES):
                        sv = jnp.full((LANES,), sv16[j], dtype=jnp.float32)
                        for q in range(QF):
                            sl = (g + j, pl.ds(q * LANES, LANES))
                            rb[sl] = rb[sl] * sv

                pltpu.async_copy(rb, acc.at[e2_c.at[1, b]], ssem, add=True)

            pltpu.make_async_copy(
                rbufs[0], acc.at[e2_c.at[1, CHUNK_BLKS - 2]], ssem).wait()
            pltpu.make_async_copy(
                rbufs[1], acc.at[e2_c.at[1, CHUNK_BLKS - 1]], ssem).wait()

        plsc.subcore_barrier()

    final = bufs[N_LAYERS % 2]
    for j in range(ROWS_PER_TILE // OUT_CHUNK):
        rj = r0 + j * OUT_CHUNK
        pltpu.sync_copy(final.at[pl.ds(rj, OUT_CHUNK)], rows)
        pltpu.sync_copy(h0_hbm.at[c, pl.ds(rj, OUT_CHUNK)], hbuf)

        @pl.loop(0, OUT_CHUNK)
        def _row(i):
            for q in range(QF):
                sl = (i, pl.ds(q * LANES, LANES))
                v = rows[sl] - hbuf[sl]
                rows[sl] = 1.0 / (1.0 + jnp.exp(-2.0 * v))

        pltpu.sync_copy(rows, out_hbm.at[c, pl.ds(rj, OUT_CHUNK)])


def kernel(h_0, edge_index, weight_tensor, layer_weights):
    n_layers, n_edges = layer_weights.shape
    pad = E_PAD - n_edges

    src = jnp.concatenate(
        [edge_index[0].astype(jnp.int32), jnp.zeros((pad,), jnp.int32)])
    dst = jnp.concatenate(
        [edge_index[1].astype(jnp.int32), jnp.zeros((pad,), jnp.int32)])
    wt = jnp.concatenate(
        [weight_tensor.astype(jnp.float32), jnp.zeros((pad,), jnp.float32)])
    lw = jnp.concatenate(
        [layer_weights.astype(jnp.float32),
         jnp.zeros((n_layers, pad), jnp.float32)], axis=1)

    blk = (NS, NCH, CHUNK_BLKS, EDGE_BLK)
    e2 = jnp.stack([src.reshape(blk), dst.reshape(blk)], axis=2)
    wl = jnp.stack([jnp.broadcast_to(wt.reshape(blk), (n_layers,) + blk),
                    lw.reshape((n_layers,) + blk)], axis=3)
    h0p = jnp.pad(h_0, ((0, N_PAD - N_NODES), (0, 0)))
    h0s = h0p.reshape(N_PAD, NC, DH).transpose(1, 0, 2)

    mesh = plsc.VectorSubcoreMesh(core_axis_name="c", subcore_axis_name="s")
    run = pl.kernel(
        _sc_body,
        out_type=jax.ShapeDtypeStruct((NC, N_PAD, DH), jnp.float32),
        mesh=mesh,
        compiler_params=pltpu.CompilerParams(use_tc_tiling_on_sc=False),
        scratch_types=[
            pltpu.VMEM_SHARED((N_PAD, DH), jnp.float32),
            pltpu.VMEM_SHARED((N_PAD, DH), jnp.float32),
            pltpu.VMEM((2, CHUNK_BLKS, EDGE_BLK), jnp.int32),
            pltpu.VMEM((2, CHUNK_BLKS, EDGE_BLK), jnp.float32),
            pltpu.VMEM((EDGE_BLK, DH), jnp.float32),
            pltpu.VMEM((EDGE_BLK, DH), jnp.float32),
            pltpu.VMEM((OUT_CHUNK, DH), jnp.float32),
            pltpu.SemaphoreType.DMA,
            pltpu.SemaphoreType.DMA,
        ],
    )
    out = run(h0s, e2, wl)
    return out.transpose(1, 0, 2).reshape(N_PAD, D_FEAT)[:N_NODES]


# speedup vs baseline: 1.1781x; 1.1781x over previous
import jax
import jax.numpy as jnp
from jax import lax
from jax.experimental import pallas as pl
from jax.experimental.pallas import tpu as pltpu
from jax.experimental.pallas import tpu_sc as plsc

N_NODES = 10000
N_PAD = 10240
D_FEAT = 128
N_LAYERS = 3

NC = 2
NS = 16
LANES = 16
DH = D_FEAT // NC
QF = DH // LANES

EDGE_BLK = 128
CHUNK_BLKS = 8
NB = 160
NCH = NB // CHUNK_BLKS
E_PAD = NS * NB * EDGE_BLK
ROWS_PER_TILE = N_PAD // NS
OUT_CHUNK = 128


def _sc_body(h0_hbm, e2_hbm, wl_hbm, out_hbm,
             h_a, h_b, e2_c, wl_c, rows, rows2, rows3, hbuf, gsem, ssem):
    c = lax.axis_index("c")
    s = lax.axis_index("s")
    r0 = s * ROWS_PER_TILE

    pltpu.sync_copy(h0_hbm.at[c, pl.ds(r0, ROWS_PER_TILE)],
                    h_a.at[pl.ds(r0, ROWS_PER_TILE)])

    bufs = [h_a, h_b]
    for k in range(N_LAYERS):
        gsrc = bufs[k % 2]
        acc = bufs[(k + 1) % 2]
        pltpu.sync_copy(h0_hbm.at[c, pl.ds(r0, ROWS_PER_TILE)],
                        acc.at[pl.ds(r0, ROWS_PER_TILE)])
        plsc.subcore_barrier()

        @pl.loop(0, NCH)
        def _chunk(ch):
            pltpu.sync_copy(e2_hbm.at[s, ch], e2_c)
            pltpu.sync_copy(wl_hbm.at[k, s, ch], wl_c)

            rbufs = [rows, rows2, rows3]
            pltpu.async_copy(gsrc.at[e2_c.at[0, 0]], rbufs[0], gsem)
            pltpu.async_copy(gsrc.at[e2_c.at[0, 1]], rbufs[1], gsem)
            for b in range(CHUNK_BLKS):
                rb = rbufs[b % 3]
                pltpu.make_async_copy(
                    gsrc.at[e2_c.at[0, b]], rb, gsem).wait()

                @pl.loop(0, EDGE_BLK, step=LANES)
                def _edge_group(g):
                    sv16 = (wl_c[0, b, pl.ds(g, LANES)] *
                            wl_c[1, b, pl.ds(g, LANES)])
                    for j in range(LANES):
                        sv = jnp.full((LANES,), sv16[j], dtype=jnp.float32)
                        for q in range(QF):
                            sl = (g + j, pl.ds(q * LANES, LANES))
                            rb[sl] = rb[sl] * sv

                if b + 2 < CHUNK_BLKS:
                    if b >= 1:
                        pltpu.make_async_copy(
                            rbufs[(b - 1) % 3],
                            acc.at[e2_c.at[1, b - 1]], ssem).wait()
                    pltpu.async_copy(
                        gsrc.at[e2_c.at[0, b + 2]], rbufs[(b - 1) % 3],
                        gsem)
                pltpu.async_copy(rb, acc.at[e2_c.at[1, b]], ssem, add=True)

            for b in (CHUNK_BLKS - 3, CHUNK_BLKS - 2, CHUNK_BLKS - 1):
                pltpu.make_async_copy(
                    rbufs[b % 3], acc.at[e2_c.at[1, b]], ssem).wait()

        plsc.subcore_barrier()

    final = bufs[N_LAYERS % 2]
    for j in range(ROWS_PER_TILE // OUT_CHUNK):
        rj = r0 + j * OUT_CHUNK
        pltpu.sync_copy(final.at[pl.ds(rj, OUT_CHUNK)], rows)
        pltpu.sync_copy(h0_hbm.at[c, pl.ds(rj, OUT_CHUNK)], hbuf)

        @pl.loop(0, OUT_CHUNK)
        def _row(i):
            for q in range(QF):
                sl = (i, pl.ds(q * LANES, LANES))
                v = rows[sl] - hbuf[sl]
                rows[sl] = 1.0 / (1.0 + jnp.exp(-2.0 * v))

        pltpu.sync_copy(rows, out_hbm.at[c, pl.ds(rj, OUT_CHUNK)])


def kernel(h_0, edge_index, weight_tensor, layer_weights):
    n_layers, n_edges = layer_weights.shape
    pad = E_PAD - n_edges

    src = jnp.concatenate(
        [edge_index[0].astype(jnp.int32), jnp.zeros((pad,), jnp.int32)])
    dst = jnp.concatenate(
        [edge_index[1].astype(jnp.int32), jnp.zeros((pad,), jnp.int32)])
    wt = jnp.concatenate(
        [weight_tensor.astype(jnp.float32), jnp.zeros((pad,), jnp.float32)])
    lw = jnp.concatenate(
        [layer_weights.astype(jnp.float32),
         jnp.zeros((n_layers, pad), jnp.float32)], axis=1)

    blk = (NS, NCH, CHUNK_BLKS, EDGE_BLK)
    e2 = jnp.stack([src.reshape(blk), dst.reshape(blk)], axis=2)
    wl = jnp.stack([jnp.broadcast_to(wt.reshape(blk), (n_layers,) + blk),
                    lw.reshape((n_layers,) + blk)], axis=3)
    h0p = jnp.pad(h_0, ((0, N_PAD - N_NODES), (0, 0)))
    h0s = h0p.reshape(N_PAD, NC, DH).transpose(1, 0, 2)

    mesh = plsc.VectorSubcoreMesh(core_axis_name="c", subcore_axis_name="s")
    run = pl.kernel(
        _sc_body,
        out_type=jax.ShapeDtypeStruct((NC, N_PAD, DH), jnp.float32),
        mesh=mesh,
        compiler_params=pltpu.CompilerParams(use_tc_tiling_on_sc=False),
        scratch_types=[
            pltpu.VMEM_SHARED((N_PAD, DH), jnp.float32),
            pltpu.VMEM_SHARED((N_PAD, DH), jnp.float32),
            pltpu.VMEM((2, CHUNK_BLKS, EDGE_BLK), jnp.int32),
            pltpu.VMEM((2, CHUNK_BLKS, EDGE_BLK), jnp.float32),
            pltpu.VMEM((EDGE_BLK, DH), jnp.float32),
            pltpu.VMEM((EDGE_BLK, DH), jnp.float32),
            pltpu.VMEM((EDGE_BLK, DH), jnp.float32),
            pltpu.VMEM((OUT_CHUNK, DH), jnp.float32),
            pltpu.SemaphoreType.DMA,
            pltpu.SemaphoreType.DMA,
        ],
    )
    out = run(h0s, e2, wl)
    return out.transpose(1, 0, 2).reshape(N_PAD, D_FEAT)[:N_NODES]


# speedup vs baseline: 1.2172x; 1.0332x over previous
import jax
import jax.numpy as jnp
from jax import lax
from jax.experimental import pallas as pl
from jax.experimental.pallas import tpu as pltpu
from jax.experimental.pallas import tpu_sc as plsc

N_NODES = 10000
N_PAD = 10240
D_FEAT = 128
N_LAYERS = 3

NC = 2
NS = 16
LANES = 16
DH = D_FEAT // NC
QF = DH // LANES

EDGE_BLK = 128
CHUNK_BLKS = 8
NB = 160
NCH = NB // CHUNK_BLKS
E_PAD = NS * NB * EDGE_BLK
ROWS_PER_TILE = N_PAD // NS
OUT_CHUNK = 128


def _sc_body(h0_hbm, e2_hbm, wl_hbm, out_hbm,
             h_a, h_b, e2_c, wl_c, rows, rows2, rows3, rows4, hbuf, gsem, ssem):
    c = lax.axis_index("c")
    s = lax.axis_index("s")
    r0 = s * ROWS_PER_TILE

    pltpu.sync_copy(h0_hbm.at[c, pl.ds(r0, ROWS_PER_TILE)],
                    h_a.at[pl.ds(r0, ROWS_PER_TILE)])

    bufs = [h_a, h_b]
    for k in range(N_LAYERS):
        gsrc = bufs[k % 2]
        acc = bufs[(k + 1) % 2]
        pltpu.sync_copy(h0_hbm.at[c, pl.ds(r0, ROWS_PER_TILE)],
                        acc.at[pl.ds(r0, ROWS_PER_TILE)])
        plsc.subcore_barrier()

        @pl.loop(0, NCH)
        def _chunk(ch):
            pltpu.sync_copy(e2_hbm.at[s, ch], e2_c)
            pltpu.sync_copy(wl_hbm.at[k, s, ch], wl_c)

            rbufs = [rows, rows2, rows3, rows4]
            pltpu.async_copy(gsrc.at[e2_c.at[0, 0]], rbufs[0], gsem)
            pltpu.async_copy(gsrc.at[e2_c.at[0, 1]], rbufs[1], gsem)
            pltpu.async_copy(gsrc.at[e2_c.at[0, 2]], rbufs[2], gsem)
            for b in range(CHUNK_BLKS):
                rb = rbufs[b % 4]
                pltpu.make_async_copy(
                    gsrc.at[e2_c.at[0, b]], rb, gsem).wait()

                @pl.loop(0, EDGE_BLK, step=LANES)
                def _edge_group(g):
                    sv16 = (wl_c[0, b, pl.ds(g, LANES)] *
                            wl_c[1, b, pl.ds(g, LANES)])
                    for j in range(LANES):
                        sv = jnp.full((LANES,), sv16[j], dtype=jnp.float32)
                        for q in range(QF):
                            sl = (g + j, pl.ds(q * LANES, LANES))
                            rb[sl] = rb[sl] * sv

                if b + 3 < CHUNK_BLKS:
                    if b >= 1:
                        pltpu.make_async_copy(
                            rbufs[(b - 1) % 4],
                            acc.at[e2_c.at[1, b - 1]], ssem).wait()
                    pltpu.async_copy(
                        gsrc.at[e2_c.at[0, b + 3]], rbufs[(b - 1) % 4],
                        gsem)
                pltpu.async_copy(rb, acc.at[e2_c.at[1, b]], ssem, add=True)

            for b in range(CHUNK_BLKS - 4, CHUNK_BLKS):
                pltpu.make_async_copy(
                    rbufs[b % 4], acc.at[e2_c.at[1, b]], ssem).wait()

        plsc.subcore_barrier()

    final = bufs[N_LAYERS % 2]
    for j in range(ROWS_PER_TILE // OUT_CHUNK):
        rj = r0 + j * OUT_CHUNK
        pltpu.sync_copy(final.at[pl.ds(rj, OUT_CHUNK)], rows)
        pltpu.sync_copy(h0_hbm.at[c, pl.ds(rj, OUT_CHUNK)], hbuf)

        @pl.loop(0, OUT_CHUNK)
        def _row(i):
            for q in range(QF):
                sl = (i, pl.ds(q * LANES, LANES))
                v = rows[sl] - hbuf[sl]
                rows[sl] = 1.0 / (1.0 + jnp.exp(-2.0 * v))

        pltpu.sync_copy(rows, out_hbm.at[c, pl.ds(rj, OUT_CHUNK)])


def kernel(h_0, edge_index, weight_tensor, layer_weights):
    n_layers, n_edges = layer_weights.shape
    pad = E_PAD - n_edges

    src = jnp.concatenate(
        [edge_index[0].astype(jnp.int32), jnp.zeros((pad,), jnp.int32)])
    dst = jnp.concatenate(
        [edge_index[1].astype(jnp.int32), jnp.zeros((pad,), jnp.int32)])
    wt = jnp.concatenate(
        [weight_tensor.astype(jnp.float32), jnp.zeros((pad,), jnp.float32)])
    lw = jnp.concatenate(
        [layer_weights.astype(jnp.float32),
         jnp.zeros((n_layers, pad), jnp.float32)], axis=1)

    blk = (NS, NCH, CHUNK_BLKS, EDGE_BLK)
    e2 = jnp.stack([src.reshape(blk), dst.reshape(blk)], axis=2)
    wl = jnp.stack([jnp.broadcast_to(wt.reshape(blk), (n_layers,) + blk),
                    lw.reshape((n_layers,) + blk)], axis=3)
    h0p = jnp.pad(h_0, ((0, N_PAD - N_NODES), (0, 0)))
    h0s = h0p.reshape(N_PAD, NC, DH).transpose(1, 0, 2)

    mesh = plsc.VectorSubcoreMesh(core_axis_name="c", subcore_axis_name="s")
    run = pl.kernel(
        _sc_body,
        out_type=jax.ShapeDtypeStruct((NC, N_PAD, DH), jnp.float32),
        mesh=mesh,
        compiler_params=pltpu.CompilerParams(use_tc_tiling_on_sc=False),
        scratch_types=[
            pltpu.VMEM_SHARED((N_PAD, DH), jnp.float32),
            pltpu.VMEM_SHARED((N_PAD, DH), jnp.float32),
            pltpu.VMEM((2, CHUNK_BLKS, EDGE_BLK), jnp.int32),
            pltpu.VMEM((2, CHUNK_BLKS, EDGE_BLK), jnp.float32),
            pltpu.VMEM((EDGE_BLK, DH), jnp.float32),
            pltpu.VMEM((EDGE_BLK, DH), jnp.float32),
            pltpu.VMEM((EDGE_BLK, DH), jnp.float32),
            pltpu.VMEM((EDGE_BLK, DH), jnp.float32),
            pltpu.VMEM((OUT_CHUNK, DH), jnp.float32),
            pltpu.SemaphoreType.DMA,
            pltpu.SemaphoreType.DMA,
        ],
    )
    out = run(h0s, e2, wl)
    return out.transpose(1, 0, 2).reshape(N_PAD, D_FEAT)[:N_NODES]
